# baseline (device time: 18263 ns/iter reference)
import jax
import jax.numpy as jnp
from jax import lax
from jax.experimental import pallas as pl
from jax.experimental.pallas import tpu as pltpu

N_DEV = 4
N_EXPERTS = 16
E_LOCAL = N_EXPERTS // N_DEV


def kernel(x, router_W, route_idx, expert_W, shared_W):
    n_tok, d_model = x.shape
    d_ff = shared_W.shape[1]
    rows = n_tok // N_DEV

    def body(x_hbm, rw_ref, idx_ref, ew_hbm, sw_hbm, out_ref,
             xv, ewv, swv, send_ref, comm_ref, in_sems, send_sems, recv_sems):
        my = lax.axis_index("i")

        cp_x = pltpu.make_async_copy(x_hbm, xv, in_sems.at[0])
        cp_ew = pltpu.make_async_copy(ew_hbm, ewv, in_sems.at[1])
        cp_sw = pltpu.make_async_copy(sw_hbm, swv, in_sems.at[2])
        cp_x.start()
        cp_ew.start()
        cp_sw.start()

        barrier = pltpu.get_barrier_semaphore()
        for off in (1, 2, 3):
            pl.semaphore_signal(
                barrier, inc=1,
                device_id=(lax.rem(my + off, N_DEV),),
                device_id_type=pl.DeviceIdType.MESH,
            )

        cp_x.wait()
        cp_ew.wait()

        eidx = lax.broadcasted_iota(jnp.int32, (rows, N_EXPERTS), 1)
        rw = rw_ref[:, :]
        wcat = ewv[:, :, :].astype(jnp.bfloat16).reshape(
            E_LOCAL * d_model, d_ff)

        def chunk_partial(c):
            xs = xv[pl.ds(c * rows, rows), :]
            scores = jnp.dot(xs, rw, preferred_element_type=jnp.float32)
            scores = scores - jnp.max(scores, axis=-1, keepdims=True)
            es = jnp.exp(scores)
            probs = es / jnp.sum(es, axis=-1, keepdims=True)
            ridx = idx_ref[pl.ds(c * rows, rows), :]
            xs_bf = xs.astype(jnp.bfloat16)
            xw = []
            for e_loc in range(E_LOCAL):
                e_glob = my * E_LOCAL + e_loc
                p_e = jnp.sum(jnp.where(eidx == e_glob, probs, 0.0),
                              axis=1, keepdims=True)
                w = jnp.where(ridx == e_glob, p_e, 0.0)
                xw.append(xs_bf * w.astype(jnp.bfloat16))
            xw = jnp.concatenate(xw, axis=1)
            part = jnp.dot(xw, wcat, preferred_element_type=jnp.float32)
            return xs_bf, part

        rdmas = []
        first = True
        for off in (2, 1, 3):
            dst = lax.rem(my + off, N_DEV)
            slot = 3 - off
            _, part = chunk_partial(dst)
            send_ref[slot] = part.astype(jnp.bfloat16)
            rdma = pltpu.make_async_remote_copy(
                src_ref=send_ref.at[slot],
                dst_ref=comm_ref.at[slot],
                send_sem=send_sems.at[slot],
                recv_sem=recv_sems.at[slot],
                device_id=(dst,),
                device_id_type=pl.DeviceIdType.MESH,
            )
            if first:
                pl.semaphore_wait(barrier, N_DEV - 1)
                first = False
            rdma.start()
            rdmas.append((rdma, slot))

        xs_bf_my, part_my = chunk_partial(my)
        cp_sw.wait()
        shared = jnp.dot(xs_bf_my, swv[:, :].astype(jnp.bfloat16),
                         preferred_element_type=jnp.float32)
        total = shared + part_my

        by_slot = {slot: rdma for rdma, slot in rdmas}
        for slot in (0, 2, 1):
            rdma = by_slot[slot]
            rdma.wait_recv()
            total = total + comm_ref[slot].astype(jnp.float32)
        out_ref[:, :] = total

        for rdma, _ in rdmas:
            rdma.wait_send()

    return pl.pallas_call(
        body,
        out_shape=jax.ShapeDtypeStruct((rows, d_ff), jnp.float32),
        in_specs=[
            pl.BlockSpec(memory_space=pl.ANY),
            pl.BlockSpec(memory_space=pltpu.VMEM),
            pl.BlockSpec(memory_space=pltpu.VMEM),
            pl.BlockSpec(memory_space=pl.ANY),
            pl.BlockSpec(memory_space=pl.ANY),
        ],
        out_specs=pl.BlockSpec(memory_space=pltpu.VMEM),
        scratch_shapes=[
            pltpu.VMEM((n_tok, d_model), jnp.float32),
            pltpu.VMEM((E_LOCAL, d_model, d_ff), jnp.float32),
            pltpu.VMEM((d_model, d_ff), jnp.float32),
            pltpu.VMEM((N_DEV - 1, rows, d_ff), jnp.bfloat16),
            pltpu.VMEM((N_DEV - 1, rows, d_ff), jnp.bfloat16),
            pltpu.SemaphoreType.DMA((3,)),
            pltpu.SemaphoreType.DMA((N_DEV - 1,)),
            pltpu.SemaphoreType.DMA((N_DEV - 1,)),
        ],
        compiler_params=pltpu.CompilerParams(collective_id=0),
    )(x, router_W, route_idx, expert_W, shared_W)
